# baseline (speedup 1.0000x reference)
import jax
import jax.numpy as jnp
from jax.experimental import pallas as pl
from jax.experimental.pallas import tpu as pltpu


def _pick_block(dim, preferred):
    for b in preferred:
        if dim % b == 0:
            return b
    return dim


def _mm_kernel(adj_ref, x_ref, o_ref):
    o_ref[...] = jnp.dot(
        adj_ref[...], x_ref[...], preferred_element_type=jnp.float32
    )


def kernel(x, adj):
    m, k = adj.shape
    _, d = x.shape
    bm = _pick_block(m, (80, 40, 16, 8))
    return pl.pallas_call(
        _mm_kernel,
        grid=(m // bm,),
        in_specs=[
            pl.BlockSpec((bm, k), lambda i: (i, 0)),
            pl.BlockSpec((k, d), lambda i: (0, 0)),
        ],
        out_specs=pl.BlockSpec((bm, d), lambda i: (i, 0)),
        out_shape=jax.ShapeDtypeStruct((m, d), jnp.float32),
        compiler_params=pltpu.CompilerParams(
            dimension_semantics=("parallel",),
        ),
    )(adj, x)


# speedup vs baseline: 1.3569x; 1.3569x over previous
import jax
import jax.numpy as jnp
from jax.experimental import pallas as pl
from jax.experimental.pallas import tpu as pltpu


def _pick_block(dim, preferred):
    for b in preferred:
        if dim % b == 0:
            return b
    return dim


def _mm2_kernel(a1_ref, a2_ref, x_ref, o_ref):
    xv = x_ref[...]
    o_ref[0] = jnp.dot(a1_ref[...], xv, preferred_element_type=jnp.float32)
    o_ref[1] = jnp.dot(a2_ref[...], xv, preferred_element_type=jnp.float32)


def _mm_kernel(adj_ref, x_ref, o_ref):
    o_ref[...] = jnp.dot(
        adj_ref[...], x_ref[...], preferred_element_type=jnp.float32
    )


def kernel(x, adj):
    m, k = adj.shape
    _, d = x.shape
    if m % 2 == 0:
        half = m // 2
        bs = _pick_block(half, (200, 40, 8))
        if bs != half:
            nsteps = half // bs
            out = pl.pallas_call(
                _mm2_kernel,
                grid=(nsteps,),
                in_specs=[
                    pl.BlockSpec((bs, k), lambda i: (i, 0)),
                    pl.BlockSpec((bs, k), lambda i, o=nsteps: (i + o, 0)),
                    pl.BlockSpec((k, d), lambda i: (0, 0)),
                ],
                out_specs=pl.BlockSpec((2, bs, d), lambda i: (0, i, 0)),
                out_shape=jax.ShapeDtypeStruct((2, half, d), jnp.float32),
                compiler_params=pltpu.CompilerParams(
                    dimension_semantics=("parallel",),
                ),
            )(adj, adj, x)
            return out.reshape(m, d)
    bm = _pick_block(m, (400, 200, 80, 40, 16, 8))
    return pl.pallas_call(
        _mm_kernel,
        grid=(m // bm,),
        in_specs=[
            pl.BlockSpec((bm, k), lambda i: (i, 0)),
            pl.BlockSpec((k, d), lambda i: (0, 0)),
        ],
        out_specs=pl.BlockSpec((bm, d), lambda i: (i, 0)),
        out_shape=jax.ShapeDtypeStruct((m, d), jnp.float32),
        compiler_params=pltpu.CompilerParams(
            dimension_semantics=("parallel",),
        ),
    )(adj, x)
